# baseline (speedup 1.0000x reference)
import functools

import jax
import jax.numpy as jnp
from jax import lax
from jax.experimental import pallas as pl
from jax.experimental.pallas import tpu as pltpu
from jax.experimental.pallas import tpu_sc as plsc

MAX_LEN = 4096
L = 16
CG = 8


def _sc_body(B, d, T, x_hbm, dur_hbm, out_hbm, mel_hbm,
             dur_v, idx_v, mask_v, xdata0, xdata1, obuf0, obuf1, mel_v,
             si0, si1, so0, so1):
    del mel_v
    cid = lax.axis_index("c")
    sid = lax.axis_index("s")
    wid = sid * 2 + cid
    b = wid % B
    h = wid // B
    half = d // 2
    c0 = h * half
    n_groups = half // CG
    n_chunks = MAX_LEN // L

    pltpu.sync_copy(dur_hbm.at[b], dur_v)
    pltpu.async_copy(x_hbm.at[b, pl.ds(c0, CG), :], xdata0, si0)
    pltpu.async_copy(x_hbm.at[b, pl.ds(c0 + CG, CG), :], xdata1, si1)

    @plsc.parallel_loop(0, n_chunks)
    def _(i):
        idx_v[pl.ds(i * L, L)] = jnp.full((L,), T, jnp.int32)

    @plsc.parallel_loop(0, T // L, carry=jnp.int32(0))
    def total(i, carry):
        dchunk = dur_v[pl.ds(i * L, L)]
        cum = plsc.cumsum(dchunk) + carry
        excl = cum - dchunk
        tvec = lax.iota(jnp.int32, L) + i * L
        for r in range(7):
            plsc.store_scatter(idx_v, [excl + r], tvec, mask=dchunk > r)
        return carry + jnp.sum(dchunk)

    nvalid = (total + (L - 1)) // L

    @plsc.parallel_loop(0, n_chunks)
    def _(i):
        ii = idx_v[pl.ds(i * L, L)]
        mask_v[pl.ds(i * L, L)] = (ii < T).astype(jnp.float32)
        idx_v[pl.ds(i * L, L)] = jnp.minimum(ii, T - 1)

    @plsc.parallel_loop(nvalid, n_chunks)
    def _(j):
        for c in range(CG):
            obuf0[c, pl.ds(j * L, L)] = jnp.zeros((L,), jnp.float32)
            obuf1[c, pl.ds(j * L, L)] = jnp.zeros((L,), jnp.float32)

    cvecs = [jnp.full((L,), c, jnp.int32) for c in range(CG)]

    def do_group(g, xd, ob, si, so):
        pltpu.make_async_copy(
            x_hbm.at[b, pl.ds(c0 + g * CG, CG), :], xd, si).wait()

        @pl.when(g >= 2)
        def _():
            pltpu.make_async_copy(
                ob, out_hbm.at[b, pl.ds(c0 + (g - 2) * CG, CG), :], so).wait()

        @plsc.parallel_loop(0, nvalid)
        def _(j):
            ii = idx_v[pl.ds(j * L, L)]
            mv = mask_v[pl.ds(j * L, L)]
            for c in range(CG):
                v = plsc.load_gather(xd, [cvecs[c], ii])
                ob[c, pl.ds(j * L, L)] = v * mv

        @pl.when(g + 2 < n_groups)
        def _():
            pltpu.async_copy(
                x_hbm.at[b, pl.ds(c0 + (g + 2) * CG, CG), :], xd, si)

        pltpu.async_copy(ob, out_hbm.at[b, pl.ds(c0 + g * CG, CG), :], so)

    def group_body(g, _):
        @pl.when(g % 2 == 0)
        def _():
            do_group(g, xdata0, obuf0, si0, so0)

        @pl.when(g % 2 == 1)
        def _():
            do_group(g, xdata1, obuf1, si1, so1)
        return 0
    lax.fori_loop(0, n_groups, group_body, 0)

    @pl.when(h == 1)
    def _():
        dur_v[pl.ds(0, L)] = jnp.broadcast_to(total, (L,))
        pltpu.sync_copy(dur_v.at[pl.ds(0, L)], mel_hbm.at[pl.ds(b * L, L)])

    pltpu.make_async_copy(
        obuf0, out_hbm.at[b, pl.ds(c0 + (n_groups - 2) * CG, CG), :], so0).wait()
    pltpu.make_async_copy(
        obuf1, out_hbm.at[b, pl.ds(c0 + (n_groups - 1) * CG, CG), :], so1).wait()


def kernel(x, duration, max_len):
    B, d, T = x.shape
    mesh = plsc.VectorSubcoreMesh(core_axis_name="c", subcore_axis_name="s")
    body = functools.partial(_sc_body, B, d, T)
    out, mel_pad = pl.kernel(
        body,
        mesh=mesh,
        compiler_params=pltpu.CompilerParams(needs_layout_passes=False),
        out_type=[
            jax.ShapeDtypeStruct((B, d, MAX_LEN), jnp.float32),
            jax.ShapeDtypeStruct((B * L,), jnp.int32),
        ],
        scratch_types=[
            pltpu.VMEM((T,), jnp.int32),
            pltpu.VMEM((MAX_LEN,), jnp.int32),
            pltpu.VMEM((MAX_LEN,), jnp.float32),
            pltpu.VMEM((CG, T), jnp.float32),
            pltpu.VMEM((CG, T), jnp.float32),
            pltpu.VMEM((CG, MAX_LEN), jnp.float32),
            pltpu.VMEM((CG, MAX_LEN), jnp.float32),
            pltpu.VMEM((B,), jnp.int32),
            pltpu.SemaphoreType.DMA,
            pltpu.SemaphoreType.DMA,
            pltpu.SemaphoreType.DMA,
            pltpu.SemaphoreType.DMA,
        ],
    )(x, duration)
    return out, mel_pad.reshape(B, L)[:, 0]


# speedup vs baseline: 1.0139x; 1.0139x over previous
import functools

import jax
import jax.numpy as jnp
from jax import lax
from jax.experimental import pallas as pl
from jax.experimental.pallas import tpu as pltpu
from jax.experimental.pallas import tpu_sc as plsc

MAX_LEN = 4096
L = 16
CG = 8


def _sc_body(B, d, T, x_hbm, dur_hbm, out_hbm, mel_hbm,
             dur_v, idx_v, mask_v, xdata0, xdata1, obuf0, obuf1, mel_v,
             si0, si1, so0, so1):
    del mel_v
    cid = lax.axis_index("c")
    sid = lax.axis_index("s")
    wid = sid * 2 + cid
    b = wid % B
    h = wid // B
    half = d // 2
    c0 = h * half
    n_groups = half // CG
    n_chunks = MAX_LEN // L

    pltpu.sync_copy(dur_hbm.at[b], dur_v)
    pltpu.async_copy(x_hbm.at[b, pl.ds(c0, CG), :], xdata0, si0)
    pltpu.async_copy(x_hbm.at[b, pl.ds(c0 + CG, CG), :], xdata1, si1)

    @plsc.parallel_loop(0, n_chunks)
    def _(i):
        idx_v[pl.ds(i * L, L)] = jnp.full((L,), T, jnp.int32)

    @plsc.parallel_loop(0, T // L, carry=jnp.int32(0))
    def total(i, carry):
        dchunk = dur_v[pl.ds(i * L, L)]
        cum = plsc.cumsum(dchunk) + carry
        excl = cum - dchunk
        tvec = lax.iota(jnp.int32, L) + i * L
        for r in range(7):
            plsc.store_scatter(idx_v, [excl + r], tvec, mask=dchunk > r)
        return carry + jnp.sum(dchunk)

    nvalid = (total + (L - 1)) // L

    @plsc.parallel_loop(0, nvalid)
    def _(i):
        ii = idx_v[pl.ds(i * L, L)]
        mask_v[pl.ds(i * L, L)] = (ii < T).astype(jnp.float32)
        idx_v[pl.ds(i * L, L)] = jnp.minimum(ii, T - 1)

    @plsc.parallel_loop(nvalid, n_chunks)
    def _(j):
        for c in range(CG):
            obuf0[c, pl.ds(j * L, L)] = jnp.zeros((L,), jnp.float32)
            obuf1[c, pl.ds(j * L, L)] = jnp.zeros((L,), jnp.float32)

    cvecs = [jnp.full((L,), c, jnp.int32) for c in range(CG)]

    def do_group(g, xd, ob, si, so):
        pltpu.make_async_copy(
            x_hbm.at[b, pl.ds(c0 + g * CG, CG), :], xd, si).wait()

        @pl.when(g >= 2)
        def _():
            pltpu.make_async_copy(
                ob, out_hbm.at[b, pl.ds(c0 + (g - 2) * CG, CG), :], so).wait()

        @plsc.parallel_loop(0, nvalid, unroll=2)
        def _(j):
            ii = idx_v[pl.ds(j * L, L)]
            mv = mask_v[pl.ds(j * L, L)]
            for c in range(CG):
                v = plsc.load_gather(xd, [cvecs[c], ii])
                ob[c, pl.ds(j * L, L)] = v * mv

        @pl.when(g + 2 < n_groups)
        def _():
            pltpu.async_copy(
                x_hbm.at[b, pl.ds(c0 + (g + 2) * CG, CG), :], xd, si)

        pltpu.async_copy(ob, out_hbm.at[b, pl.ds(c0 + g * CG, CG), :], so)

    def group_body(g, _):
        @pl.when(g % 2 == 0)
        def _():
            do_group(g, xdata0, obuf0, si0, so0)

        @pl.when(g % 2 == 1)
        def _():
            do_group(g, xdata1, obuf1, si1, so1)
        return 0
    lax.fori_loop(0, n_groups, group_body, 0)

    @pl.when(h == 1)
    def _():
        dur_v[pl.ds(0, L)] = jnp.broadcast_to(total, (L,))
        pltpu.sync_copy(dur_v.at[pl.ds(0, L)], mel_hbm.at[pl.ds(b * L, L)])

    pltpu.make_async_copy(
        obuf0, out_hbm.at[b, pl.ds(c0 + (n_groups - 2) * CG, CG), :], so0).wait()
    pltpu.make_async_copy(
        obuf1, out_hbm.at[b, pl.ds(c0 + (n_groups - 1) * CG, CG), :], so1).wait()


def kernel(x, duration, max_len):
    B, d, T = x.shape
    mesh = plsc.VectorSubcoreMesh(core_axis_name="c", subcore_axis_name="s")
    body = functools.partial(_sc_body, B, d, T)
    out, mel_pad = pl.kernel(
        body,
        mesh=mesh,
        compiler_params=pltpu.CompilerParams(needs_layout_passes=False),
        out_type=[
            jax.ShapeDtypeStruct((B, d, MAX_LEN), jnp.float32),
            jax.ShapeDtypeStruct((B * L,), jnp.int32),
        ],
        scratch_types=[
            pltpu.VMEM((T,), jnp.int32),
            pltpu.VMEM((MAX_LEN,), jnp.int32),
            pltpu.VMEM((MAX_LEN,), jnp.float32),
            pltpu.VMEM((CG, T), jnp.float32),
            pltpu.VMEM((CG, T), jnp.float32),
            pltpu.VMEM((CG, MAX_LEN), jnp.float32),
            pltpu.VMEM((CG, MAX_LEN), jnp.float32),
            pltpu.VMEM((B,), jnp.int32),
            pltpu.SemaphoreType.DMA,
            pltpu.SemaphoreType.DMA,
            pltpu.SemaphoreType.DMA,
            pltpu.SemaphoreType.DMA,
        ],
    )(x, duration)
    return out, mel_pad.reshape(B, L)[:, 0]


# speedup vs baseline: 1.0385x; 1.0243x over previous
import functools

import jax
import jax.numpy as jnp
from jax import lax
from jax.experimental import pallas as pl
from jax.experimental.pallas import tpu as pltpu
from jax.experimental.pallas import tpu_sc as plsc

MAX_LEN = 4096
L = 16
CG = 8


def _sc_body(B, d, T, x_hbm, dur_hbm, out_hbm, mel_hbm,
             dur_v, idx_v, mask_v, xdata0, xdata1, obuf0, obuf1, mel_v,
             si0, si1, so0, so1):
    del mel_v
    cid = lax.axis_index("c")
    sid = lax.axis_index("s")
    wid = sid * 2 + cid
    b = wid % B
    h = wid // B
    half = d // 2
    c0 = h * half
    n_groups = half // CG
    n_chunks = MAX_LEN // L

    pltpu.sync_copy(dur_hbm.at[b], dur_v)
    pltpu.async_copy(x_hbm.at[b, pl.ds(c0, CG), :], xdata0, si0)
    pltpu.async_copy(x_hbm.at[b, pl.ds(c0 + CG, CG), :], xdata1, si1)

    @plsc.parallel_loop(0, T // L, carry=jnp.int32(0))
    def total(i, carry):
        dchunk = dur_v[pl.ds(i * L, L)]
        cum = plsc.cumsum(dchunk) + carry
        excl = cum - dchunk
        tvec = lax.iota(jnp.int32, L) + i * L
        for r in range(7):
            plsc.store_scatter(idx_v, [excl + r], tvec, mask=dchunk > r)
        return carry + jnp.sum(dchunk)

    nvalid = (total + (L - 1)) // L

    @plsc.parallel_loop(0, nvalid)
    def _(i):
        ii = idx_v[pl.ds(i * L, L)]
        pos = lax.iota(jnp.int32, L) + i * L
        mask_v[pl.ds(i * L, L)] = (pos < total).astype(jnp.float32)
        idx_v[pl.ds(i * L, L)] = jnp.clip(ii, 0, T - 1)

    @plsc.parallel_loop(nvalid, n_chunks)
    def _(j):
        for c in range(CG):
            obuf0[c, pl.ds(j * L, L)] = jnp.zeros((L,), jnp.float32)
            obuf1[c, pl.ds(j * L, L)] = jnp.zeros((L,), jnp.float32)

    cvecs = [jnp.full((L,), c, jnp.int32) for c in range(CG)]

    def do_group(g, xd, ob, si, so):
        pltpu.make_async_copy(
            x_hbm.at[b, pl.ds(c0 + g * CG, CG), :], xd, si).wait()

        @pl.when(g >= 2)
        def _():
            pltpu.make_async_copy(
                ob, out_hbm.at[b, pl.ds(c0 + (g - 2) * CG, CG), :], so).wait()

        @plsc.parallel_loop(0, nvalid, unroll=2)
        def _(j):
            ii = idx_v[pl.ds(j * L, L)]
            mv = mask_v[pl.ds(j * L, L)]
            for c in range(CG):
                v = plsc.load_gather(xd, [cvecs[c], ii])
                ob[c, pl.ds(j * L, L)] = v * mv

        @pl.when(g + 2 < n_groups)
        def _():
            pltpu.async_copy(
                x_hbm.at[b, pl.ds(c0 + (g + 2) * CG, CG), :], xd, si)

        pltpu.async_copy(ob, out_hbm.at[b, pl.ds(c0 + g * CG, CG), :], so)

    def group_body(g, _):
        @pl.when(g % 2 == 0)
        def _():
            do_group(g, xdata0, obuf0, si0, so0)

        @pl.when(g % 2 == 1)
        def _():
            do_group(g, xdata1, obuf1, si1, so1)
        return 0
    lax.fori_loop(0, n_groups, group_body, 0)

    @pl.when(h == 1)
    def _():
        dur_v[pl.ds(0, L)] = jnp.broadcast_to(total, (L,))
        pltpu.sync_copy(dur_v.at[pl.ds(0, L)], mel_hbm.at[pl.ds(b * L, L)])

    pltpu.make_async_copy(
        obuf0, out_hbm.at[b, pl.ds(c0 + (n_groups - 2) * CG, CG), :], so0).wait()
    pltpu.make_async_copy(
        obuf1, out_hbm.at[b, pl.ds(c0 + (n_groups - 1) * CG, CG), :], so1).wait()


def kernel(x, duration, max_len):
    B, d, T = x.shape
    mesh = plsc.VectorSubcoreMesh(core_axis_name="c", subcore_axis_name="s")
    body = functools.partial(_sc_body, B, d, T)
    out, mel_pad = pl.kernel(
        body,
        mesh=mesh,
        compiler_params=pltpu.CompilerParams(needs_layout_passes=False),
        out_type=[
            jax.ShapeDtypeStruct((B, d, MAX_LEN), jnp.float32),
            jax.ShapeDtypeStruct((B * L,), jnp.int32),
        ],
        scratch_types=[
            pltpu.VMEM((T,), jnp.int32),
            pltpu.VMEM((MAX_LEN,), jnp.int32),
            pltpu.VMEM((MAX_LEN,), jnp.float32),
            pltpu.VMEM((CG, T), jnp.float32),
            pltpu.VMEM((CG, T), jnp.float32),
            pltpu.VMEM((CG, MAX_LEN), jnp.float32),
            pltpu.VMEM((CG, MAX_LEN), jnp.float32),
            pltpu.VMEM((B,), jnp.int32),
            pltpu.SemaphoreType.DMA,
            pltpu.SemaphoreType.DMA,
            pltpu.SemaphoreType.DMA,
            pltpu.SemaphoreType.DMA,
        ],
    )(x, duration)
    return out, mel_pad.reshape(B, L)[:, 0]
